# baseline (speedup 1.0000x reference)
import functools

import jax
import jax.numpy as jnp
from jax.experimental import pallas as pl
from jax.experimental.pallas import tpu as pltpu

N = 4096
D = 128
H = 64
BLK = 512
NB = N // BLK


def _gnn_kernel(x_ref, adj_ref, wc0, wc1, wc2, b0, b1, b2,
                out_ref, adj_bf, g_s, root_s):
    l = pl.program_id(0)
    i = pl.program_id(1)
    cur = jax.lax.rem(l, 2)
    nxt = 1 - cur

    @pl.when(jnp.logical_and(l == 0, i == 0))
    def _():
        t = jax.lax.dot_general(x_ref[...], wc0[...],
                                (((1,), (0,)), ((), ())),
                                preferred_element_type=jnp.float32)
        g_s[0] = t[:, :H].astype(jnp.bfloat16)
        root_s[0] = t[:, H:]

    def step(a, bias, wnext, last):
        agg = jax.lax.dot_general(a, g_s[cur],
                                  (((0,), (0,)), ((), ())),
                                  preferred_element_type=jnp.float32)
        res = jnp.maximum(agg + root_s[cur, pl.ds(i * BLK, BLK), :] + bias,
                          0.0)
        out_ref[...] = res
        if not last:
            t = jax.lax.dot_general(res, wnext,
                                    (((1,), (0,)), ((), ())),
                                    preferred_element_type=jnp.float32)
            g_s[nxt, pl.ds(i * BLK, BLK), :] = t[:, :H].astype(jnp.bfloat16)
            root_s[nxt, pl.ds(i * BLK, BLK), :] = t[:, H:]

    @pl.when(l == 0)
    def _():
        a = adj_ref[...].astype(jnp.bfloat16)
        adj_bf[i] = a
        step(a, b0[...], wc1[...], last=False)

    @pl.when(l == 1)
    def _():
        step(adj_bf[i], b1[...], wc2[...], last=False)

    @pl.when(l == 2)
    def _():
        step(adj_bf[i], b2[...], None, last=True)


@functools.partial(jax.jit, static_argnames=("interpret",))
def _run(X, adj_mat, W_rel0, b_rel0, W_root0, W_rel1, b_rel1, W_root1,
         W_rel2, b_rel2, W_root2, interpret=False):
    wc0 = jnp.concatenate([W_rel0.T, W_root0.T], axis=1)
    wc1 = jnp.concatenate([W_rel1.T, W_root1.T], axis=1)
    wc2 = jnp.concatenate([W_rel2.T, W_root2.T], axis=1)
    b0 = b_rel0.reshape(1, H)
    b1 = b_rel1.reshape(1, H)
    b2 = b_rel2.reshape(1, H)
    full = lambda shape: pl.BlockSpec(shape, lambda l, i: (0,) * len(shape))
    return pl.pallas_call(
        _gnn_kernel,
        grid=(3, NB),
        in_specs=[
            full((N, D)),
            pl.BlockSpec((N, BLK),
                         lambda l, i: (0, jnp.where(l == 0, i, 0))),
            full((D, 2 * H)), full((H, 2 * H)), full((H, 2 * H)),
            full((1, H)), full((1, H)), full((1, H)),
        ],
        out_specs=pl.BlockSpec((BLK, H), lambda l, i: (i, 0)),
        out_shape=jax.ShapeDtypeStruct((N, H), jnp.float32),
        scratch_shapes=[
            pltpu.VMEM((NB, N, BLK), jnp.bfloat16),
            pltpu.VMEM((2, N, H), jnp.bfloat16),
            pltpu.VMEM((2, N, H), jnp.float32),
        ],
        interpret=interpret,
    )(X, adj_mat, wc0, wc1, wc2, b0, b1, b2)


def kernel(X, adj_mat, W_rel0, b_rel0, W_root0, W_rel1, b_rel1, W_root1,
           W_rel2, b_rel2, W_root2):
    return _run(X, adj_mat, W_rel0, b_rel0, W_root0, W_rel1, b_rel1, W_root1,
                W_rel2, b_rel2, W_root2)
